# baseline (speedup 1.0000x reference)
import functools

import jax
import jax.numpy as jnp
from jax import lax
from jax.experimental import pallas as pl
from jax.experimental.pallas import tpu as pltpu
from jax.experimental.pallas import tpu_sc as plsc

TOPK_K = 30
B, T, L = 64, 50, 8192
LANES = 16
NV = L // LANES
UNROLL = 8
NUM_CORES = 2
NUM_SUBCORES = 16
NW = NUM_CORES * NUM_SUBCORES
B_PER_W = B // NW
MAX_ROW = B * T - 1

RADIX_SHIFTS = (28, 24, 20, 16, 12, 8, 4, 0)


def _i32(x):
  return x - (1 << 32) if x >= (1 << 31) else x


HIMASKS = [_i32((0xFFFFFFFF << (s + 4)) & 0xFFFFFFFF) for s in RADIX_SHIFTS]


@jax.jit
def kernel(input):
  x_flat = input.reshape(-1)
  mesh = plsc.VectorSubcoreMesh(core_axis_name="c", subcore_axis_name="s")

  @functools.partial(
      pl.kernel,
      out_type=jax.ShapeDtypeStruct((B * T * L,), jnp.float32),
      mesh=mesh,
      scratch_types=[
          pltpu.VMEM((L,), jnp.float32),
          pltpu.VMEM((L,), jnp.float32),
          pltpu.VMEM((L,), jnp.float32),
          pltpu.VMEM((L,), jnp.float32),
          pltpu.VMEM((L,), jnp.float32),
          pltpu.VMEM((L,), jnp.int32),
          pltpu.VMEM((L,), jnp.int32),
          pltpu.VMEM((LANES,), jnp.int32),
          pltpu.VMEM((LANES,), jnp.int32),
          pltpu.SemaphoreType.DMA,
          pltpu.SemaphoreType.DMA,
          pltpu.SemaphoreType.DMA,
          pltpu.SemaphoreType.DMA,
      ],
      compiler_params=pltpu.CompilerParams(needs_layout_passes=False),
  )
  def k(x_hbm, o_hbm, in0, in1, out0, out1, carry_v, cand_v, cidx_v,
        hist_v, suf_v, si0, si1, so0, so1):
    wid = lax.axis_index("s") * NUM_CORES + lax.axis_index("c")
    zero_f = jnp.zeros((LANES,), jnp.float32)
    zero_i = jnp.zeros((LANES,), jnp.int32)
    ones_i = jnp.ones((LANES,), jnp.int32)
    lane_iota = lax.iota(jnp.int32, LANES)

    def in_slice(b, t):
      r = jnp.minimum(b * T + t, MAX_ROW) * L
      return x_hbm.at[pl.ds(r, L)]

    def out_slice(b, t):
      return o_hbm.at[pl.ds((b * T + t) * L, L)]

    def step(b, t, theta, in_v, out_v):
      def acc_body(j, off_vec):
        for u in range(UNROLL):
          base = (j * UNROLL + u) * LANES
          sl = pl.ds(base, LANES)
          cv = carry_v[sl] + in_v[sl]
          carry_v[sl] = cv
          m = cv >= theta
          out_v[sl] = jnp.where(m, cv, jnp.float32(0.0))

          @pl.when(jnp.any(m))
          def _():
            pos = off_vec + plsc.cumsum(m.astype(jnp.int32)) - 1
            plsc.store_scatter(cand_v, [pos], plsc.bitcast(cv, jnp.int32),
                               mask=m)
            plsc.store_scatter(cidx_v, [pos], lane_iota + base, mask=m)

          off_vec = off_vec + plsc.all_reduce_population_count(m)
        return off_vec

      c_vec = lax.fori_loop(0, NV // UNROLL, acc_body, zero_i)
      c = jnp.max(c_vec)
      nvc = lax.shift_right_logical(c + (LANES - 1), 4)

      need_vec = jnp.full((LANES,), TOPK_K, jnp.int32)
      prefix_vec = zero_i
      for shift, himask in zip(RADIX_SHIFTS, HIMASKS):
        hist_v[pl.ds(0, LANES)] = zero_i

        def fill_body(j, _, shift=shift, himask=himask,
                      prefix_vec=prefix_vec):
          bits = cand_v[pl.ds(j * LANES, LANES)]
          lane_ok = (j * LANES + lane_iota) < c_vec
          pref_ok = (bits & jnp.int32(himask)) == prefix_vec
          digit = lax.shift_right_logical(bits, jnp.int32(shift)) & 0xF
          plsc.addupdate_scatter(hist_v, [digit], ones_i,
                                 mask=jnp.logical_and(lane_ok, pref_ok))
          return 0

        lax.fori_loop(0, nvc, fill_body, 0)
        h = hist_v[pl.ds(0, LANES)]
        cs = plsc.cumsum(lax.rev(h, (0,)))
        suf_v[pl.ds(0, LANES)] = lax.rev(cs, (0,))
        d_vec = plsc.all_reduce_population_count(cs >= need_vec) - 1
        idx = jnp.minimum(d_vec + 1, LANES - 1)
        above = plsc.load_gather(suf_v, [idx])
        count_above = jnp.where(d_vec >= LANES - 1, zero_i, above)
        need_vec = need_vec - count_above
        prefix_vec = prefix_vec | lax.shift_left(d_vec, jnp.int32(shift))

      def corr_body(j, _):
        bits = cand_v[pl.ds(j * LANES, LANES)]
        idxs = cidx_v[pl.ds(j * LANES, LANES)]
        lane_ok = (j * LANES + lane_iota) < c_vec
        bad = jnp.logical_and(lane_ok, bits < prefix_vec)
        plsc.store_scatter(out_v, [idxs], zero_f, mask=bad)
        return 0

      lax.fori_loop(0, nvc, corr_body, 0)
      return plsc.bitcast(prefix_vec, jnp.float32)

    for bi in range(B_PER_W):
      b = wid * B_PER_W + bi

      def zc_body(j, _):
        for u in range(UNROLL):
          carry_v[pl.ds((j * UNROLL + u) * LANES, LANES)] = zero_f
        return 0

      lax.fori_loop(0, NV // UNROLL, zc_body, 0)

      pltpu.async_copy(in_slice(b, 0), in0, si0)
      pltpu.async_copy(in_slice(b, 1), in1, si1)
      pltpu.make_async_copy(in_slice(b, 0), in0, si0).wait()
      theta = step(b, 0, zero_f, in0, out0)
      pltpu.async_copy(out0, out_slice(b, 0), so0)
      pltpu.async_copy(in_slice(b, 2), in0, si0)
      pltpu.make_async_copy(in_slice(b, 1), in1, si1).wait()
      theta = step(b, 1, theta, in1, out1)
      pltpu.async_copy(out1, out_slice(b, 1), so1)

      def dt_body(i, theta):
        t0 = 2 * i
        pltpu.async_copy(in_slice(b, t0 + 1), in1, si1)
        pltpu.make_async_copy(in_slice(b, t0), in0, si0).wait()
        pltpu.make_async_copy(out0, out_slice(b, t0 - 2), so0).wait()
        theta = step(b, t0, theta, in0, out0)
        pltpu.async_copy(out0, out_slice(b, t0), so0)
        pltpu.async_copy(in_slice(b, t0 + 2), in0, si0)
        pltpu.make_async_copy(in_slice(b, t0 + 1), in1, si1).wait()
        pltpu.make_async_copy(out1, out_slice(b, t0 - 1), so1).wait()
        theta = step(b, t0 + 1, theta, in1, out1)
        pltpu.async_copy(out1, out_slice(b, t0 + 1), so1)
        return theta

      lax.fori_loop(1, T // 2, dt_body, theta)

      pltpu.make_async_copy(in_slice(b, T), in0, si0).wait()
      pltpu.make_async_copy(out0, out_slice(b, T - 2), so0).wait()
      pltpu.make_async_copy(out1, out_slice(b, T - 1), so1).wait()

  return k(x_flat).reshape(input.shape)


# speedup vs baseline: 1.5729x; 1.5729x over previous
import functools

import jax
import jax.numpy as jnp
from jax import lax
from jax.experimental import pallas as pl
from jax.experimental.pallas import tpu as pltpu
from jax.experimental.pallas import tpu_sc as plsc

TOPK_K = 30
B, T, L = 64, 50, 8192
LANES = 16
NV = L // LANES
UNROLL = 8
NUM_CORES = 2
NUM_SUBCORES = 16
NW = NUM_CORES * NUM_SUBCORES
B_PER_W = B // NW
MAX_ROW = B * T - 1

RADIX_SHIFTS = (28, 24, 20, 16, 12, 8, 4, 0)


def _i32(x):
  return x - (1 << 32) if x >= (1 << 31) else x


HIMASKS = [_i32((0xFFFFFFFF << (s + 4)) & 0xFFFFFFFF) for s in RADIX_SHIFTS]


@jax.jit
def kernel(input):
  x_flat = input.reshape(-1)
  mesh = plsc.VectorSubcoreMesh(core_axis_name="c", subcore_axis_name="s")

  @functools.partial(
      pl.kernel,
      out_type=jax.ShapeDtypeStruct((B * T * L,), jnp.float32),
      mesh=mesh,
      scratch_types=[
          pltpu.VMEM((L,), jnp.float32),
          pltpu.VMEM((L,), jnp.float32),
          pltpu.VMEM((L,), jnp.float32),
          pltpu.VMEM((L,), jnp.float32),
          pltpu.VMEM((L,), jnp.float32),
          pltpu.VMEM((L,), jnp.int32),
          pltpu.VMEM((L,), jnp.int32),
          pltpu.VMEM((LANES,), jnp.int32),
          pltpu.VMEM((LANES,), jnp.int32),
          pltpu.SemaphoreType.DMA,
          pltpu.SemaphoreType.DMA,
          pltpu.SemaphoreType.DMA,
          pltpu.SemaphoreType.DMA,
      ],
      compiler_params=pltpu.CompilerParams(needs_layout_passes=False),
  )
  def k(x_hbm, o_hbm, in0, in1, out0, out1, carry_v, cand_v, cidx_v,
        hist_v, suf_v, si0, si1, so0, so1):
    wid = lax.axis_index("s") * NUM_CORES + lax.axis_index("c")
    zero_f = jnp.zeros((LANES,), jnp.float32)
    zero_i = jnp.zeros((LANES,), jnp.int32)
    ones_i = jnp.ones((LANES,), jnp.int32)
    lane_iota = lax.iota(jnp.int32, LANES)

    def in_slice(b, t):
      r = jnp.minimum(b * T + t, MAX_ROW) * L
      return x_hbm.at[pl.ds(r, L)]

    def out_slice(b, t):
      return o_hbm.at[pl.ds((b * T + t) * L, L)]

    def step(b, t, theta, in_v, out_v):
      def acc_body(j, off_vec):
        for u in range(UNROLL):
          base = (j * UNROLL + u) * LANES
          sl = pl.ds(base, LANES)
          cv = carry_v[sl] + in_v[sl]
          carry_v[sl] = cv
          m = cv >= theta
          out_v[sl] = jnp.where(m, cv, jnp.float32(0.0))
          pos = off_vec + plsc.cumsum(m.astype(jnp.int32)) - 1
          plsc.store_scatter(cand_v, [pos], plsc.bitcast(cv, jnp.int32),
                             mask=m)
          plsc.store_scatter(cidx_v, [pos], lane_iota + base, mask=m)
          off_vec = off_vec + plsc.all_reduce_population_count(m)
        return off_vec

      c_vec = lax.fori_loop(0, NV // UNROLL, acc_body, zero_i)
      c = jnp.max(c_vec)
      nvc = lax.shift_right_logical(c + (LANES - 1), 4)

      need_vec = jnp.full((LANES,), TOPK_K, jnp.int32)
      prefix_vec = zero_i
      for shift, himask in zip(RADIX_SHIFTS, HIMASKS):
        hist_v[pl.ds(0, LANES)] = zero_i

        def fill_body(j, _, shift=shift, himask=himask,
                      prefix_vec=prefix_vec):
          bits = cand_v[pl.ds(j * LANES, LANES)]
          lane_ok = (j * LANES + lane_iota) < c_vec
          pref_ok = (bits & jnp.int32(himask)) == prefix_vec
          digit = lax.shift_right_logical(bits, jnp.int32(shift)) & 0xF
          plsc.addupdate_scatter(hist_v, [digit], ones_i,
                                 mask=jnp.logical_and(lane_ok, pref_ok))
          return 0

        lax.fori_loop(0, nvc, fill_body, 0)
        h = hist_v[pl.ds(0, LANES)]
        cs = plsc.cumsum(lax.rev(h, (0,)))
        suf_v[pl.ds(0, LANES)] = lax.rev(cs, (0,))
        d_vec = plsc.all_reduce_population_count(cs >= need_vec) - 1
        idx = jnp.minimum(d_vec + 1, LANES - 1)
        above = plsc.load_gather(suf_v, [idx])
        count_above = jnp.where(d_vec >= LANES - 1, zero_i, above)
        need_vec = need_vec - count_above
        prefix_vec = prefix_vec | lax.shift_left(d_vec, jnp.int32(shift))

      def corr_body(j, _):
        bits = cand_v[pl.ds(j * LANES, LANES)]
        idxs = cidx_v[pl.ds(j * LANES, LANES)]
        lane_ok = (j * LANES + lane_iota) < c_vec
        bad = jnp.logical_and(lane_ok, bits < prefix_vec)
        plsc.store_scatter(out_v, [idxs], zero_f, mask=bad)
        return 0

      lax.fori_loop(0, nvc, corr_body, 0)
      return plsc.bitcast(prefix_vec, jnp.float32)

    for bi in range(B_PER_W):
      b = wid * B_PER_W + bi

      def zc_body(j, _):
        for u in range(UNROLL):
          carry_v[pl.ds((j * UNROLL + u) * LANES, LANES)] = zero_f
        return 0

      lax.fori_loop(0, NV // UNROLL, zc_body, 0)

      pltpu.async_copy(in_slice(b, 0), in0, si0)
      pltpu.async_copy(in_slice(b, 1), in1, si1)
      pltpu.make_async_copy(in_slice(b, 0), in0, si0).wait()
      theta = step(b, 0, zero_f, in0, out0)
      pltpu.async_copy(out0, out_slice(b, 0), so0)
      pltpu.async_copy(in_slice(b, 2), in0, si0)
      pltpu.make_async_copy(in_slice(b, 1), in1, si1).wait()
      theta = step(b, 1, theta, in1, out1)
      pltpu.async_copy(out1, out_slice(b, 1), so1)

      def dt_body(i, theta):
        t0 = 2 * i
        pltpu.async_copy(in_slice(b, t0 + 1), in1, si1)
        pltpu.make_async_copy(in_slice(b, t0), in0, si0).wait()
        pltpu.make_async_copy(out0, out_slice(b, t0 - 2), so0).wait()
        theta = step(b, t0, theta, in0, out0)
        pltpu.async_copy(out0, out_slice(b, t0), so0)
        pltpu.async_copy(in_slice(b, t0 + 2), in0, si0)
        pltpu.make_async_copy(in_slice(b, t0 + 1), in1, si1).wait()
        pltpu.make_async_copy(out1, out_slice(b, t0 - 1), so1).wait()
        theta = step(b, t0 + 1, theta, in1, out1)
        pltpu.async_copy(out1, out_slice(b, t0 + 1), so1)
        return theta

      lax.fori_loop(1, T // 2, dt_body, theta)

      pltpu.make_async_copy(in_slice(b, T), in0, si0).wait()
      pltpu.make_async_copy(out0, out_slice(b, T - 2), so0).wait()
      pltpu.make_async_copy(out1, out_slice(b, T - 1), so1).wait()

  return k(x_flat).reshape(input.shape)


# speedup vs baseline: 1.5868x; 1.0088x over previous
import functools

import jax
import jax.numpy as jnp
from jax import lax
from jax.experimental import pallas as pl
from jax.experimental.pallas import tpu as pltpu
from jax.experimental.pallas import tpu_sc as plsc

TOPK_K = 30
B, T, L = 64, 50, 8192
LANES = 16
NV = L // LANES
UNROLL = 8
NUM_CORES = 2
NUM_SUBCORES = 16
NW = NUM_CORES * NUM_SUBCORES
B_PER_W = B // NW
MAX_ROW = B * T - 1

RADIX_SHIFTS = (28, 24, 20, 16, 12, 8, 4, 0)


def _i32(x):
  return x - (1 << 32) if x >= (1 << 31) else x


HIMASKS = [_i32((0xFFFFFFFF << (s + 4)) & 0xFFFFFFFF) for s in RADIX_SHIFTS]


@jax.jit
def kernel(input):
  x_flat = input.reshape(-1)
  mesh = plsc.VectorSubcoreMesh(core_axis_name="c", subcore_axis_name="s")

  @functools.partial(
      pl.kernel,
      out_type=jax.ShapeDtypeStruct((B * T * L,), jnp.float32),
      mesh=mesh,
      scratch_types=[
          pltpu.VMEM((L,), jnp.float32),
          pltpu.VMEM((L,), jnp.float32),
          pltpu.VMEM((L,), jnp.float32),
          pltpu.VMEM((L,), jnp.float32),
          pltpu.VMEM((L,), jnp.float32),
          pltpu.VMEM((L + LANES,), jnp.int32),
          pltpu.VMEM((L,), jnp.int32),
          pltpu.VMEM((LANES,), jnp.int32),
          pltpu.VMEM((LANES,), jnp.int32),
          pltpu.SemaphoreType.DMA,
          pltpu.SemaphoreType.DMA,
          pltpu.SemaphoreType.DMA,
          pltpu.SemaphoreType.DMA,
      ],
      compiler_params=pltpu.CompilerParams(needs_layout_passes=False),
  )
  def k(x_hbm, o_hbm, in0, in1, out0, out1, carry_v, cand_v, cidx_v,
        hist_v, suf_v, si0, si1, so0, so1):
    wid = lax.axis_index("s") * NUM_CORES + lax.axis_index("c")
    zero_f = jnp.zeros((LANES,), jnp.float32)
    zero_i = jnp.zeros((LANES,), jnp.int32)
    ones_i = jnp.ones((LANES,), jnp.int32)
    lane_iota = lax.iota(jnp.int32, LANES)

    def in_slice(b, t):
      r = jnp.minimum(b * T + t, MAX_ROW) * L
      return x_hbm.at[pl.ds(r, L)]

    def out_slice(b, t):
      return o_hbm.at[pl.ds((b * T + t) * L, L)]

    def step(b, t, theta, in_v, out_v):
      def acc_body(j, blk_vec):
        for u in range(UNROLL):
          base = (j * UNROLL + u) * LANES
          sl = pl.ds(base, LANES)
          cv = carry_v[sl] + in_v[sl]
          carry_v[sl] = cv
          m = cv >= theta
          out_v[sl] = jnp.where(m, cv, jnp.float32(0.0))
          entry = jnp.where(m, lane_iota + base, jnp.int32(-1))
          pos = lax.shift_left(blk_vec, 4) + lane_iota
          plsc.store_scatter(cand_v, [pos], entry)
          cnt = plsc.all_reduce_population_count(m)
          blk_vec = blk_vec + jnp.minimum(cnt, ones_i)
        return blk_vec

      blk_vec = lax.fori_loop(0, NV // UNROLL, acc_body, zero_i)
      plsc.store_scatter(cand_v, [lax.shift_left(blk_vec, 4) + lane_iota],
                         jnp.full((LANES,), -1, jnp.int32))
      nblk = jnp.max(blk_vec) + 1

      def dc_body(j, off_vec):
        entry = cand_v[pl.ds(j * LANES, LANES)]
        valid = entry >= 0
        pos = off_vec + plsc.cumsum(valid.astype(jnp.int32)) - 1
        plsc.store_scatter(cidx_v, [pos], entry, mask=valid)
        return off_vec + plsc.all_reduce_population_count(valid)

      c_vec = lax.fori_loop(0, nblk, dc_body, zero_i)
      c = jnp.max(c_vec)
      nvc = lax.shift_right_logical(c + (LANES - 1), 4)

      need_vec = jnp.full((LANES,), TOPK_K, jnp.int32)
      prefix_vec = zero_i
      for shift, himask in zip(RADIX_SHIFTS, HIMASKS):
        hist_v[pl.ds(0, LANES)] = zero_i

        def fill_body(j, _, shift=shift, himask=himask,
                      prefix_vec=prefix_vec):
          idxs = cidx_v[pl.ds(j * LANES, LANES)]
          idxc = jnp.clip(idxs, 0, L - 1)
          bits = plsc.bitcast(plsc.load_gather(carry_v, [idxc]), jnp.int32)
          lane_ok = (j * LANES + lane_iota) < c_vec
          pref_ok = (bits & jnp.int32(himask)) == prefix_vec
          digit = lax.shift_right_logical(bits, jnp.int32(shift)) & 0xF
          plsc.addupdate_scatter(hist_v, [digit], ones_i,
                                 mask=jnp.logical_and(lane_ok, pref_ok))
          return 0

        lax.fori_loop(0, nvc, fill_body, 0)
        h = hist_v[pl.ds(0, LANES)]
        cs = plsc.cumsum(lax.rev(h, (0,)))
        suf_v[pl.ds(0, LANES)] = lax.rev(cs, (0,))
        d_vec = plsc.all_reduce_population_count(cs >= need_vec) - 1
        idx = jnp.minimum(d_vec + 1, LANES - 1)
        above = plsc.load_gather(suf_v, [idx])
        count_above = jnp.where(d_vec >= LANES - 1, zero_i, above)
        need_vec = need_vec - count_above
        prefix_vec = prefix_vec | lax.shift_left(d_vec, jnp.int32(shift))

      def corr_body(j, _):
        idxs = cidx_v[pl.ds(j * LANES, LANES)]
        idxc = jnp.clip(idxs, 0, L - 1)
        bits = plsc.bitcast(plsc.load_gather(carry_v, [idxc]), jnp.int32)
        lane_ok = (j * LANES + lane_iota) < c_vec
        bad = jnp.logical_and(lane_ok, bits < prefix_vec)
        plsc.store_scatter(out_v, [idxc], zero_f, mask=bad)
        return 0

      lax.fori_loop(0, nvc, corr_body, 0)
      return plsc.bitcast(prefix_vec, jnp.float32)

    for bi in range(B_PER_W):
      b = wid * B_PER_W + bi

      def zc_body(j, _):
        for u in range(UNROLL):
          carry_v[pl.ds((j * UNROLL + u) * LANES, LANES)] = zero_f
        return 0

      lax.fori_loop(0, NV // UNROLL, zc_body, 0)

      pltpu.async_copy(in_slice(b, 0), in0, si0)
      pltpu.async_copy(in_slice(b, 1), in1, si1)
      pltpu.make_async_copy(in_slice(b, 0), in0, si0).wait()
      theta = step(b, 0, zero_f, in0, out0)
      pltpu.async_copy(out0, out_slice(b, 0), so0)
      pltpu.async_copy(in_slice(b, 2), in0, si0)
      pltpu.make_async_copy(in_slice(b, 1), in1, si1).wait()
      theta = step(b, 1, theta, in1, out1)
      pltpu.async_copy(out1, out_slice(b, 1), so1)

      def dt_body(i, theta):
        t0 = 2 * i
        pltpu.async_copy(in_slice(b, t0 + 1), in1, si1)
        pltpu.make_async_copy(in_slice(b, t0), in0, si0).wait()
        pltpu.make_async_copy(out0, out_slice(b, t0 - 2), so0).wait()
        theta = step(b, t0, theta, in0, out0)
        pltpu.async_copy(out0, out_slice(b, t0), so0)
        pltpu.async_copy(in_slice(b, t0 + 2), in0, si0)
        pltpu.make_async_copy(in_slice(b, t0 + 1), in1, si1).wait()
        pltpu.make_async_copy(out1, out_slice(b, t0 - 1), so1).wait()
        theta = step(b, t0 + 1, theta, in1, out1)
        pltpu.async_copy(out1, out_slice(b, t0 + 1), so1)
        return theta

      lax.fori_loop(1, T // 2, dt_body, theta)

      pltpu.make_async_copy(in_slice(b, T), in0, si0).wait()
      pltpu.make_async_copy(out0, out_slice(b, T - 2), so0).wait()
      pltpu.make_async_copy(out1, out_slice(b, T - 1), so1).wait()

  return k(x_flat).reshape(input.shape)


# speedup vs baseline: 2.2458x; 1.4153x over previous
import functools

import jax
import jax.numpy as jnp
from jax import lax
from jax.experimental import pallas as pl
from jax.experimental.pallas import tpu as pltpu
from jax.experimental.pallas import tpu_sc as plsc

TOPK_K = 30
B, T, L = 64, 50, 8192
LANES = 16
NV = L // LANES
UNROLL = 8
NUM_CORES = 2
NUM_SUBCORES = 16
NW = NUM_CORES * NUM_SUBCORES
B_PER_W = B // NW
MAX_ROW = B * T - 1

RADIX_SHIFTS = (28, 24, 20, 16, 12, 8, 4, 0)


def _i32(x):
  return x - (1 << 32) if x >= (1 << 31) else x


HIMASKS = [_i32((0xFFFFFFFF << (s + 4)) & 0xFFFFFFFF) for s in RADIX_SHIFTS]


@jax.jit
def kernel(input):
  x_flat = input.reshape(-1)
  mesh = plsc.VectorSubcoreMesh(core_axis_name="c", subcore_axis_name="s")

  @functools.partial(
      pl.kernel,
      out_type=jax.ShapeDtypeStruct((B * T * L,), jnp.float32),
      mesh=mesh,
      scratch_types=[
          pltpu.VMEM((L,), jnp.float32),
          pltpu.VMEM((L,), jnp.float32),
          pltpu.VMEM((L,), jnp.float32),
          pltpu.VMEM((L,), jnp.float32),
          pltpu.VMEM((L,), jnp.float32),
          pltpu.VMEM((L,), jnp.int32),
          pltpu.VMEM((LANES,), jnp.int32),
          pltpu.VMEM((LANES,), jnp.int32),
          pltpu.SemaphoreType.DMA,
          pltpu.SemaphoreType.DMA,
          pltpu.SemaphoreType.DMA,
          pltpu.SemaphoreType.DMA,
      ],
      compiler_params=pltpu.CompilerParams(needs_layout_passes=False),
  )
  def k(x_hbm, o_hbm, in0, in1, out0, out1, carry_v, cidx_v,
        hist_v, suf_v, si0, si1, so0, so1):
    wid = lax.axis_index("s") * NUM_CORES + lax.axis_index("c")
    zero_f = jnp.zeros((LANES,), jnp.float32)
    zero_i = jnp.zeros((LANES,), jnp.int32)
    ones_i = jnp.ones((LANES,), jnp.int32)
    lane_iota = lax.iota(jnp.int32, LANES)

    def in_slice(b, t):
      r = jnp.minimum(b * T + t, MAX_ROW) * L
      return x_hbm.at[pl.ds(r, L)]

    def out_slice(b, t):
      return o_hbm.at[pl.ds((b * T + t) * L, L)]

    def step(b, t, theta, in_v, out_v):
      @plsc.parallel_loop(0, NV, step=1, unroll=UNROLL)
      def _(j):
        sl = pl.ds(j * LANES, LANES)
        cv = carry_v[sl] + in_v[sl]
        carry_v[sl] = cv
        out_v[sl] = jnp.where(cv >= theta, cv, jnp.float32(0.0))

      @plsc.parallel_loop(0, NV, step=1, unroll=UNROLL, carry=zero_i)
      def c_vec(j, off_vec):
        base = j * LANES
        valid = carry_v[pl.ds(base, LANES)] >= theta
        pos = off_vec + plsc.cumsum(valid.astype(jnp.int32)) - 1
        plsc.store_scatter(cidx_v, [pos], lane_iota + base, mask=valid)
        return off_vec + plsc.all_reduce_population_count(valid)

      c = jnp.max(c_vec)
      nvc = lax.shift_right_logical(c + (LANES - 1), 4)

      need_vec = jnp.full((LANES,), TOPK_K, jnp.int32)
      prefix_vec = zero_i
      for shift, himask in zip(RADIX_SHIFTS, HIMASKS):
        hist_v[pl.ds(0, LANES)] = zero_i

        def fill_body(j, _, shift=shift, himask=himask,
                      prefix_vec=prefix_vec):
          idxs = cidx_v[pl.ds(j * LANES, LANES)]
          idxc = jnp.clip(idxs, 0, L - 1)
          bits = plsc.bitcast(plsc.load_gather(carry_v, [idxc]), jnp.int32)
          lane_ok = (j * LANES + lane_iota) < c_vec
          pref_ok = (bits & jnp.int32(himask)) == prefix_vec
          digit = lax.shift_right_logical(bits, jnp.int32(shift)) & 0xF
          plsc.addupdate_scatter(hist_v, [digit], ones_i,
                                 mask=jnp.logical_and(lane_ok, pref_ok))
          return 0

        lax.fori_loop(0, nvc, fill_body, 0)
        h = hist_v[pl.ds(0, LANES)]
        cs = plsc.cumsum(lax.rev(h, (0,)))
        suf_v[pl.ds(0, LANES)] = lax.rev(cs, (0,))
        d_vec = plsc.all_reduce_population_count(cs >= need_vec) - 1
        idx = jnp.minimum(d_vec + 1, LANES - 1)
        above = plsc.load_gather(suf_v, [idx])
        count_above = jnp.where(d_vec >= LANES - 1, zero_i, above)
        need_vec = need_vec - count_above
        prefix_vec = prefix_vec | lax.shift_left(d_vec, jnp.int32(shift))

      def corr_body(j, _):
        idxs = cidx_v[pl.ds(j * LANES, LANES)]
        idxc = jnp.clip(idxs, 0, L - 1)
        bits = plsc.bitcast(plsc.load_gather(carry_v, [idxc]), jnp.int32)
        lane_ok = (j * LANES + lane_iota) < c_vec
        bad = jnp.logical_and(lane_ok, bits < prefix_vec)
        plsc.store_scatter(out_v, [idxc], zero_f, mask=bad)
        return 0

      lax.fori_loop(0, nvc, corr_body, 0)
      return plsc.bitcast(prefix_vec, jnp.float32)

    for bi in range(B_PER_W):
      b = wid * B_PER_W + bi

      def zc_body(j, _):
        for u in range(UNROLL):
          carry_v[pl.ds((j * UNROLL + u) * LANES, LANES)] = zero_f
        return 0

      lax.fori_loop(0, NV // UNROLL, zc_body, 0)

      pltpu.async_copy(in_slice(b, 0), in0, si0)
      pltpu.async_copy(in_slice(b, 1), in1, si1)
      pltpu.make_async_copy(in_slice(b, 0), in0, si0).wait()
      theta = step(b, 0, zero_f, in0, out0)
      pltpu.async_copy(out0, out_slice(b, 0), so0)
      pltpu.async_copy(in_slice(b, 2), in0, si0)
      pltpu.make_async_copy(in_slice(b, 1), in1, si1).wait()
      theta = step(b, 1, theta, in1, out1)
      pltpu.async_copy(out1, out_slice(b, 1), so1)

      def dt_body(i, theta):
        t0 = 2 * i
        pltpu.async_copy(in_slice(b, t0 + 1), in1, si1)
        pltpu.make_async_copy(in_slice(b, t0), in0, si0).wait()
        pltpu.make_async_copy(out0, out_slice(b, t0 - 2), so0).wait()
        theta = step(b, t0, theta, in0, out0)
        pltpu.async_copy(out0, out_slice(b, t0), so0)
        pltpu.async_copy(in_slice(b, t0 + 2), in0, si0)
        pltpu.make_async_copy(in_slice(b, t0 + 1), in1, si1).wait()
        pltpu.make_async_copy(out1, out_slice(b, t0 - 1), so1).wait()
        theta = step(b, t0 + 1, theta, in1, out1)
        pltpu.async_copy(out1, out_slice(b, t0 + 1), so1)
        return theta

      lax.fori_loop(1, T // 2, dt_body, theta)

      pltpu.make_async_copy(in_slice(b, T), in0, si0).wait()
      pltpu.make_async_copy(out0, out_slice(b, T - 2), so0).wait()
      pltpu.make_async_copy(out1, out_slice(b, T - 1), so1).wait()

  return k(x_flat).reshape(input.shape)


# speedup vs baseline: 3.3592x; 1.4958x over previous
import functools

import jax
import jax.numpy as jnp
from jax import lax
from jax.experimental import pallas as pl
from jax.experimental.pallas import tpu as pltpu
from jax.experimental.pallas import tpu_sc as plsc

TOPK_K = 30
B, T, L = 64, 50, 8192
LANES = 16
NV = L // LANES
UNROLL = 8
NUM_CORES = 2
NUM_SUBCORES = 16
NW = NUM_CORES * NUM_SUBCORES
B_PER_W = B // NW
MAX_ROW = B * T - 1

RADIX_SHIFTS = (28, 24, 20, 16, 12, 8, 4, 0)


def _i32(x):
  return x - (1 << 32) if x >= (1 << 31) else x


HIMASKS = [_i32((0xFFFFFFFF << (s + 4)) & 0xFFFFFFFF) for s in RADIX_SHIFTS]


@jax.jit
def kernel(input):
  x_flat = input.reshape(-1)
  mesh = plsc.VectorSubcoreMesh(core_axis_name="c", subcore_axis_name="s")

  @functools.partial(
      pl.kernel,
      out_type=jax.ShapeDtypeStruct((B * T * L,), jnp.float32),
      mesh=mesh,
      scratch_types=[
          pltpu.VMEM((L,), jnp.float32),
          pltpu.VMEM((L,), jnp.float32),
          pltpu.VMEM((L,), jnp.float32),
          pltpu.VMEM((L,), jnp.float32),
          pltpu.VMEM((L,), jnp.float32),
          pltpu.VMEM((L,), jnp.int32),
          pltpu.VMEM((LANES,), jnp.int32),
          pltpu.VMEM((LANES,), jnp.int32),
          pltpu.SemaphoreType.DMA,
          pltpu.SemaphoreType.DMA,
          pltpu.SemaphoreType.DMA,
          pltpu.SemaphoreType.DMA,
      ],
      compiler_params=pltpu.CompilerParams(needs_layout_passes=False),
  )
  def k(x_hbm, o_hbm, in0, in1, out0, out1, carry_v, cidx_v,
        hist_v, suf_v, si0, si1, so0, so1):
    wid = lax.axis_index("s") * NUM_CORES + lax.axis_index("c")
    zero_f = jnp.zeros((LANES,), jnp.float32)
    zero_i = jnp.zeros((LANES,), jnp.int32)
    ones_i = jnp.ones((LANES,), jnp.int32)
    lane_iota = lax.iota(jnp.int32, LANES)

    def in_slice(b, t):
      r = jnp.minimum(b * T + t, MAX_ROW) * L
      return x_hbm.at[pl.ds(r, L)]

    def out_slice(b, t):
      return o_hbm.at[pl.ds((b * T + t) * L, L)]

    def step(b, t, theta, in_v, out_v):
      @plsc.parallel_loop(0, NV, step=1, unroll=UNROLL, carry=zero_i)
      def c_vec(j, off_vec):
        base = j * LANES
        sl = pl.ds(base, LANES)
        cv = carry_v[sl] + in_v[sl]
        carry_v[sl] = cv
        m = cv >= theta
        out_v[sl] = jnp.where(m, cv, jnp.float32(0.0))
        pos = off_vec + plsc.cumsum(m.astype(jnp.int32)) - 1
        plsc.store_scatter(cidx_v, [pos], lane_iota + base, mask=m)
        return off_vec + plsc.all_reduce_population_count(m)

      c = jnp.max(c_vec)
      nvc = lax.shift_right_logical(c + (LANES - 1), 4)

      need_vec = jnp.full((LANES,), TOPK_K, jnp.int32)
      prefix_vec = zero_i
      for shift, himask in zip(RADIX_SHIFTS, HIMASKS):
        hist_v[pl.ds(0, LANES)] = zero_i

        @plsc.parallel_loop(0, nvc, step=1)
        def _(j, shift=shift, himask=himask, prefix_vec=prefix_vec):
          idxs = cidx_v[pl.ds(j * LANES, LANES)]
          idxc = jnp.clip(idxs, 0, L - 1)
          bits = plsc.bitcast(plsc.load_gather(carry_v, [idxc]), jnp.int32)
          lane_ok = (j * LANES + lane_iota) < c_vec
          pref_ok = (bits & jnp.int32(himask)) == prefix_vec
          digit = lax.shift_right_logical(bits, jnp.int32(shift)) & 0xF
          plsc.addupdate_scatter(hist_v, [digit], ones_i,
                                 mask=jnp.logical_and(lane_ok, pref_ok))
        h = hist_v[pl.ds(0, LANES)]
        cs = plsc.cumsum(lax.rev(h, (0,)))
        suf_v[pl.ds(0, LANES)] = lax.rev(cs, (0,))
        d_vec = plsc.all_reduce_population_count(cs >= need_vec) - 1
        idx = jnp.minimum(d_vec + 1, LANES - 1)
        above = plsc.load_gather(suf_v, [idx])
        count_above = jnp.where(d_vec >= LANES - 1, zero_i, above)
        need_vec = need_vec - count_above
        prefix_vec = prefix_vec | lax.shift_left(d_vec, jnp.int32(shift))

      @plsc.parallel_loop(0, nvc, step=1)
      def _(j, prefix_vec=prefix_vec):
        idxs = cidx_v[pl.ds(j * LANES, LANES)]
        idxc = jnp.clip(idxs, 0, L - 1)
        bits = plsc.bitcast(plsc.load_gather(carry_v, [idxc]), jnp.int32)
        lane_ok = (j * LANES + lane_iota) < c_vec
        bad = jnp.logical_and(lane_ok, bits < prefix_vec)
        plsc.store_scatter(out_v, [idxc], zero_f, mask=bad)

      return plsc.bitcast(prefix_vec, jnp.float32)

    for bi in range(B_PER_W):
      b = wid * B_PER_W + bi

      @plsc.parallel_loop(0, NV, step=1, unroll=UNROLL)
      def _(j):
        carry_v[pl.ds(j * LANES, LANES)] = zero_f

      pltpu.async_copy(in_slice(b, 0), in0, si0)
      pltpu.async_copy(in_slice(b, 1), in1, si1)
      pltpu.make_async_copy(in_slice(b, 0), in0, si0).wait()
      theta = step(b, 0, zero_f, in0, out0)
      pltpu.async_copy(out0, out_slice(b, 0), so0)
      pltpu.async_copy(in_slice(b, 2), in0, si0)
      pltpu.make_async_copy(in_slice(b, 1), in1, si1).wait()
      theta = step(b, 1, theta, in1, out1)
      pltpu.async_copy(out1, out_slice(b, 1), so1)

      def dt_body(i, theta):
        t0 = 2 * i
        pltpu.async_copy(in_slice(b, t0 + 1), in1, si1)
        pltpu.make_async_copy(in_slice(b, t0), in0, si0).wait()
        pltpu.make_async_copy(out0, out_slice(b, t0 - 2), so0).wait()
        theta = step(b, t0, theta, in0, out0)
        pltpu.async_copy(out0, out_slice(b, t0), so0)
        pltpu.async_copy(in_slice(b, t0 + 2), in0, si0)
        pltpu.make_async_copy(in_slice(b, t0 + 1), in1, si1).wait()
        pltpu.make_async_copy(out1, out_slice(b, t0 - 1), so1).wait()
        theta = step(b, t0 + 1, theta, in1, out1)
        pltpu.async_copy(out1, out_slice(b, t0 + 1), so1)
        return theta

      lax.fori_loop(1, T // 2, dt_body, theta)

      pltpu.make_async_copy(in_slice(b, T), in0, si0).wait()
      pltpu.make_async_copy(out0, out_slice(b, T - 2), so0).wait()
      pltpu.make_async_copy(out1, out_slice(b, T - 1), so1).wait()

  return k(x_flat).reshape(input.shape)


# speedup vs baseline: 4.8407x; 1.4410x over previous
import functools

import jax
import jax.numpy as jnp
from jax import lax
from jax.experimental import pallas as pl
from jax.experimental.pallas import tpu as pltpu
from jax.experimental.pallas import tpu_sc as plsc

TOPK_K = 30
B, T, L = 64, 50, 8192
LANES = 16
NV = L // LANES
UNROLL = 8
NUM_CORES = 2
NUM_SUBCORES = 16
NW = NUM_CORES * NUM_SUBCORES
B_PER_W = B // NW
MAX_ROW = B * T - 1

RADIX_SHIFTS = (28, 24, 20, 16, 12, 8, 4, 0)


def _i32(x):
  return x - (1 << 32) if x >= (1 << 31) else x


HIMASKS = [_i32((0xFFFFFFFF << (s + 4)) & 0xFFFFFFFF) for s in RADIX_SHIFTS]


@jax.jit
def kernel(input):
  mesh = plsc.VectorSubcoreMesh(core_axis_name="c", subcore_axis_name="s")

  @functools.partial(
      pl.kernel,
      out_type=jax.ShapeDtypeStruct((B, T, L), jnp.float32),
      mesh=mesh,
      scratch_types=[
          pltpu.VMEM((L,), jnp.float32),
          pltpu.VMEM((L,), jnp.float32),
          pltpu.VMEM((L,), jnp.float32),
          pltpu.VMEM((L,), jnp.float32),
          pltpu.VMEM((L,), jnp.float32),
          pltpu.VMEM((L,), jnp.int32),
          pltpu.VMEM((LANES,), jnp.int32),
          pltpu.VMEM((LANES,), jnp.int32),
          pltpu.SemaphoreType.DMA,
          pltpu.SemaphoreType.DMA,
          pltpu.SemaphoreType.DMA,
          pltpu.SemaphoreType.DMA,
      ],
      compiler_params=pltpu.CompilerParams(needs_layout_passes=False,
                                           use_tc_tiling_on_sc=True),
  )
  def k(x_hbm, o_hbm, in0, in1, out0, out1, carry_v, cidx_v,
        hist_v, suf_v, si0, si1, so0, so1):
    wid = lax.axis_index("s") * NUM_CORES + lax.axis_index("c")
    zero_f = jnp.zeros((LANES,), jnp.float32)
    zero_i = jnp.zeros((LANES,), jnp.int32)
    ones_i = jnp.ones((LANES,), jnp.int32)
    lane_iota = lax.iota(jnp.int32, LANES)

    def in_slice(b, t):
      return x_hbm.at[b, jnp.minimum(t, T - 1)]

    def out_slice(b, t):
      return o_hbm.at[b, t]

    def step(b, t, theta, in_v, out_v):
      @plsc.parallel_loop(0, NV, step=1, unroll=UNROLL, carry=zero_i)
      def c_vec(j, off_vec):
        base = j * LANES
        sl = pl.ds(base, LANES)
        cv = carry_v[sl] + in_v[sl]
        carry_v[sl] = cv
        m = cv >= theta
        out_v[sl] = jnp.where(m, cv, jnp.float32(0.0))
        pos = off_vec + plsc.cumsum(m.astype(jnp.int32)) - 1
        plsc.store_scatter(cidx_v, [pos], lane_iota + base, mask=m)
        return off_vec + plsc.all_reduce_population_count(m)

      c = jnp.max(c_vec)
      nvc = lax.shift_right_logical(c + (LANES - 1), 4)

      need_vec = jnp.full((LANES,), TOPK_K, jnp.int32)
      prefix_vec = zero_i
      for shift, himask in zip(RADIX_SHIFTS, HIMASKS):
        hist_v[pl.ds(0, LANES)] = zero_i

        @plsc.parallel_loop(0, nvc, step=1)
        def _(j, shift=shift, himask=himask, prefix_vec=prefix_vec):
          idxs = cidx_v[pl.ds(j * LANES, LANES)]
          idxc = jnp.clip(idxs, 0, L - 1)
          bits = plsc.bitcast(plsc.load_gather(carry_v, [idxc]), jnp.int32)
          lane_ok = (j * LANES + lane_iota) < c_vec
          pref_ok = (bits & jnp.int32(himask)) == prefix_vec
          digit = lax.shift_right_logical(bits, jnp.int32(shift)) & 0xF
          plsc.addupdate_scatter(hist_v, [digit], ones_i,
                                 mask=jnp.logical_and(lane_ok, pref_ok))
        h = hist_v[pl.ds(0, LANES)]
        cs = plsc.cumsum(lax.rev(h, (0,)))
        suf_v[pl.ds(0, LANES)] = lax.rev(cs, (0,))
        d_vec = plsc.all_reduce_population_count(cs >= need_vec) - 1
        idx = jnp.minimum(d_vec + 1, LANES - 1)
        above = plsc.load_gather(suf_v, [idx])
        count_above = jnp.where(d_vec >= LANES - 1, zero_i, above)
        need_vec = need_vec - count_above
        prefix_vec = prefix_vec | lax.shift_left(d_vec, jnp.int32(shift))

      @plsc.parallel_loop(0, nvc, step=1)
      def _(j, prefix_vec=prefix_vec):
        idxs = cidx_v[pl.ds(j * LANES, LANES)]
        idxc = jnp.clip(idxs, 0, L - 1)
        bits = plsc.bitcast(plsc.load_gather(carry_v, [idxc]), jnp.int32)
        lane_ok = (j * LANES + lane_iota) < c_vec
        bad = jnp.logical_and(lane_ok, bits < prefix_vec)
        plsc.store_scatter(out_v, [idxc], zero_f, mask=bad)

      return plsc.bitcast(prefix_vec, jnp.float32)

    for bi in range(B_PER_W):
      b = wid * B_PER_W + bi

      @plsc.parallel_loop(0, NV, step=1, unroll=UNROLL)
      def _(j):
        carry_v[pl.ds(j * LANES, LANES)] = zero_f

      pltpu.async_copy(in_slice(b, 0), in0, si0)
      pltpu.async_copy(in_slice(b, 1), in1, si1)
      pltpu.make_async_copy(in_slice(b, 0), in0, si0).wait()
      theta = step(b, 0, zero_f, in0, out0)
      pltpu.async_copy(out0, out_slice(b, 0), so0)
      pltpu.async_copy(in_slice(b, 2), in0, si0)
      pltpu.make_async_copy(in_slice(b, 1), in1, si1).wait()
      theta = step(b, 1, theta, in1, out1)
      pltpu.async_copy(out1, out_slice(b, 1), so1)

      def dt_body(i, theta):
        t0 = 2 * i
        pltpu.async_copy(in_slice(b, t0 + 1), in1, si1)
        pltpu.make_async_copy(in_slice(b, t0), in0, si0).wait()
        pltpu.make_async_copy(out0, out_slice(b, t0 - 2), so0).wait()
        theta = step(b, t0, theta, in0, out0)
        pltpu.async_copy(out0, out_slice(b, t0), so0)
        pltpu.async_copy(in_slice(b, t0 + 2), in0, si0)
        pltpu.make_async_copy(in_slice(b, t0 + 1), in1, si1).wait()
        pltpu.make_async_copy(out1, out_slice(b, t0 - 1), so1).wait()
        theta = step(b, t0 + 1, theta, in1, out1)
        pltpu.async_copy(out1, out_slice(b, t0 + 1), so1)
        return theta

      lax.fori_loop(1, T // 2, dt_body, theta)

      pltpu.make_async_copy(in_slice(b, T), in0, si0).wait()
      pltpu.make_async_copy(out0, out_slice(b, T - 2), so0).wait()
      pltpu.make_async_copy(out1, out_slice(b, T - 1), so1).wait()

  return k(input)
